# baseline (speedup 1.0000x reference)
import functools
import jax
import jax.numpy as jnp
from jax import lax
from jax.experimental import pallas as pl
from jax.experimental.pallas import tpu as pltpu
from jax.experimental.pallas import tpu_sc as plsc

B = 1024
K1 = 1024
D = 128
V = 1000000
T = 0.07

NW = 32
CH = 128
G = 2
BG = B // G


def _sc_gather(memory, idx_flat):
    n_idx = idx_flat.shape[0]
    PER_W = n_idx // NW
    NCH = PER_W // CH
    mesh = plsc.VectorSubcoreMesh(core_axis_name="c", subcore_axis_name="s")

    @functools.partial(
        pl.kernel,
        mesh=mesh,
        out_type=jax.ShapeDtypeStruct((n_idx, D), jnp.float32),
        scratch_types=[
            pltpu.VMEM((PER_W,), jnp.int32),
            pltpu.VMEM((CH, D), jnp.float32),
            pltpu.VMEM((CH, D), jnp.float32),
            pltpu.VMEM((CH, D), jnp.float32),
            pltpu.VMEM((CH, D), jnp.float32),
            pltpu.SemaphoreType.DMA,
            pltpu.SemaphoreType.DMA,
            pltpu.SemaphoreType.DMA,
            pltpu.SemaphoreType.DMA,
            pltpu.SemaphoreType.DMA,
            pltpu.SemaphoreType.DMA,
            pltpu.SemaphoreType.DMA,
            pltpu.SemaphoreType.DMA,
        ],
        compiler_params=pltpu.CompilerParams(needs_layout_passes=False),
    )
    def k(mem_hbm, idx_hbm, out_hbm, idxall,
          rows0, rows1, rows2, rows3,
          gs0, gs1, gs2, gs3, ws0, ws1, ws2, ws3):
        wid = lax.axis_index("s") * 2 + lax.axis_index("c")
        base = wid * PER_W

        pltpu.sync_copy(idx_hbm.at[pl.ds(base, PER_W)], idxall)

        NBUF = 4
        bufs = [(rows0, gs0, ws0), (rows1, gs1, ws1),
                (rows2, gs2, ws2), (rows3, gs3, ws3)]

        def issue_gather(c, rows, gsem):
            pltpu.async_copy(
                mem_hbm.at[idxall.at[pl.ds(c * CH, CH)]], rows, gsem)

        def issue_write(c, rows, wsem):
            pltpu.async_copy(rows, out_hbm.at[pl.ds(base + c * CH, CH)], wsem)

        def drain_gather(rows, gsem):
            pltpu.make_async_copy(
                mem_hbm.at[pl.ds(0, CH)], rows, gsem).wait()

        def drain_write(c, rows, wsem):
            pltpu.make_async_copy(
                rows, out_hbm.at[pl.ds(base + c * CH, CH)], wsem).wait()

        for p in range(NBUF - 1):
            issue_gather(p, bufs[p][0], bufs[p][1])

        def step(c, p):
            rows_pre, gs_pre, ws_pre = bufs[(p + NBUF - 1) % NBUF]
            rows_cur, gs_cur, ws_cur = bufs[p]

            @pl.when(c >= 1)
            def _():
                drain_write(c - 1, rows_pre, ws_pre)

            @pl.when(c + NBUF - 1 < NCH)
            def _():
                issue_gather(c + NBUF - 1, rows_pre, gs_pre)

            drain_gather(rows_cur, gs_cur)
            issue_write(c, rows_cur, ws_cur)

        def body(cg, _):
            for p in range(NBUF):
                step(cg * NBUF + p, p)
            return 0

        lax.fori_loop(0, NCH // NBUF, body, 0, unroll=False)

        pl_last = (NCH - 1) % NBUF
        drain_write(NCH - 1, bufs[pl_last][0], bufs[pl_last][2])

    return k(memory, idx_flat)


def _tc_matvec(weight, x):
    GB = 8
    nb = x.shape[0]

    def body(w_ref, x_ref, o_ref):
        w = w_ref[...]
        xb = x_ref[...]
        res = jax.lax.dot_general(
            xb, w, (((1,), (1,)), ((), ())),
            preferred_element_type=jnp.float32)
        for g in range(GB):
            o_ref[g:g + 1, :] = res[g:g + 1, g * K1:(g + 1) * K1]

    return pl.pallas_call(
        body,
        grid=(nb // GB,),
        in_specs=[
            pl.BlockSpec((GB * K1, D), lambda i: (i, 0)),
            pl.BlockSpec((GB, D), lambda i: (i, 0)),
        ],
        out_specs=pl.BlockSpec((GB, K1), lambda i: (i, 0)),
        out_shape=jax.ShapeDtypeStruct((nb, K1), jnp.float32),
    )(weight, x)


def _tc_exp_z(raw):
    RB = 128

    def body(r_ref, e_ref, z_ref):
        i = pl.program_id(0)

        @pl.when(i == 0)
        def _():
            z_ref[...] = jnp.zeros_like(z_ref)

        e = jnp.exp(r_ref[...] * (1.0 / T))
        e_ref[...] = e
        z_ref[...] += jnp.sum(e)

    return pl.pallas_call(
        body,
        grid=(B // RB,),
        in_specs=[pl.BlockSpec((RB, K1), lambda i: (i, 0))],
        out_specs=[
            pl.BlockSpec((RB, K1), lambda i: (i, 0)),
            pl.BlockSpec((8, 128), lambda i: (0, 0)),
        ],
        out_shape=[
            jax.ShapeDtypeStruct((B, K1), jnp.float32),
            jax.ShapeDtypeStruct((8, 128), jnp.float32),
        ],
    )(raw)


def _tc_normalize(expout, zsplat):
    def body(e_ref, z_ref, o_ref):
        total = z_ref[0, 0]
        scale = (B * K1) / (total * V)
        o_ref[...] = e_ref[...] * scale

    return pl.pallas_call(
        body,
        grid=(8,),
        in_specs=[
            pl.BlockSpec((B // 8, K1), lambda i: (i, 0)),
            pl.BlockSpec((8, 128), lambda i: (0, 0)),
        ],
        out_specs=pl.BlockSpec((B // 8, K1), lambda i: (i, 0)),
        out_shape=jax.ShapeDtypeStruct((B, K1), jnp.float32),
    )(expout, zsplat)


def kernel(x, memory, y, idx):
    idx = idx.at[:, 0].set(y)
    idx_flat = idx.reshape(-1)
    raws = []
    for g in range(G):
        w_g = _sc_gather(memory, idx_flat[g * BG * K1:(g + 1) * BG * K1])
        raws.append(_tc_matvec(w_g, x[g * BG:(g + 1) * BG]))
    raw = jnp.concatenate(raws, axis=0)
    expout, zsplat = _tc_exp_z(raw)
    return _tc_normalize(expout, zsplat)


# speedup vs baseline: 1.0018x; 1.0018x over previous
import functools
import jax
import jax.numpy as jnp
from jax import lax
from jax.experimental import pallas as pl
from jax.experimental.pallas import tpu as pltpu
from jax.experimental.pallas import tpu_sc as plsc

B = 1024
K1 = 1024
D = 128
V = 1000000
T = 0.07

NW = 32
CH = 128
G = 4
BG = B // G


def _sc_gather(memory, idx_flat):
    n_idx = idx_flat.shape[0]
    PER_W = n_idx // NW
    NCH = PER_W // CH
    mesh = plsc.VectorSubcoreMesh(core_axis_name="c", subcore_axis_name="s")

    @functools.partial(
        pl.kernel,
        mesh=mesh,
        out_type=jax.ShapeDtypeStruct((n_idx, D), jnp.float32),
        scratch_types=[
            pltpu.VMEM((PER_W,), jnp.int32),
            pltpu.VMEM((CH, D), jnp.float32),
            pltpu.VMEM((CH, D), jnp.float32),
            pltpu.VMEM((CH, D), jnp.float32),
            pltpu.VMEM((CH, D), jnp.float32),
            pltpu.SemaphoreType.DMA,
            pltpu.SemaphoreType.DMA,
            pltpu.SemaphoreType.DMA,
            pltpu.SemaphoreType.DMA,
            pltpu.SemaphoreType.DMA,
            pltpu.SemaphoreType.DMA,
            pltpu.SemaphoreType.DMA,
            pltpu.SemaphoreType.DMA,
        ],
        compiler_params=pltpu.CompilerParams(needs_layout_passes=False),
    )
    def k(mem_hbm, idx_hbm, out_hbm, idxall,
          rows0, rows1, rows2, rows3,
          gs0, gs1, gs2, gs3, ws0, ws1, ws2, ws3):
        wid = lax.axis_index("s") * 2 + lax.axis_index("c")
        base = wid * PER_W

        pltpu.sync_copy(idx_hbm.at[pl.ds(base, PER_W)], idxall)

        NBUF = 4
        bufs = [(rows0, gs0, ws0), (rows1, gs1, ws1),
                (rows2, gs2, ws2), (rows3, gs3, ws3)]

        def issue_gather(c, rows, gsem):
            pltpu.async_copy(
                mem_hbm.at[idxall.at[pl.ds(c * CH, CH)]], rows, gsem)

        def issue_write(c, rows, wsem):
            pltpu.async_copy(rows, out_hbm.at[pl.ds(base + c * CH, CH)], wsem)

        def drain_gather(rows, gsem):
            pltpu.make_async_copy(
                mem_hbm.at[pl.ds(0, CH)], rows, gsem).wait()

        def drain_write(c, rows, wsem):
            pltpu.make_async_copy(
                rows, out_hbm.at[pl.ds(base + c * CH, CH)], wsem).wait()

        for p in range(NBUF - 1):
            issue_gather(p, bufs[p][0], bufs[p][1])

        def step(c, p):
            rows_pre, gs_pre, ws_pre = bufs[(p + NBUF - 1) % NBUF]
            rows_cur, gs_cur, ws_cur = bufs[p]

            @pl.when(c >= 1)
            def _():
                drain_write(c - 1, rows_pre, ws_pre)

            @pl.when(c + NBUF - 1 < NCH)
            def _():
                issue_gather(c + NBUF - 1, rows_pre, gs_pre)

            drain_gather(rows_cur, gs_cur)
            issue_write(c, rows_cur, ws_cur)

        def body(cg, _):
            for p in range(NBUF):
                step(cg * NBUF + p, p)
            return 0

        lax.fori_loop(0, NCH // NBUF, body, 0, unroll=False)

        pl_last = (NCH - 1) % NBUF
        drain_write(NCH - 1, bufs[pl_last][0], bufs[pl_last][2])

    return k(memory, idx_flat)


def _tc_matvec(weight, x):
    GB = 8
    nb = x.shape[0]

    def body(w_ref, x_ref, o_ref):
        w = w_ref[...]
        xb = x_ref[...]
        res = jax.lax.dot_general(
            xb, w, (((1,), (1,)), ((), ())),
            preferred_element_type=jnp.float32)
        for g in range(GB):
            o_ref[g:g + 1, :] = res[g:g + 1, g * K1:(g + 1) * K1]

    return pl.pallas_call(
        body,
        grid=(nb // GB,),
        in_specs=[
            pl.BlockSpec((GB * K1, D), lambda i: (i, 0)),
            pl.BlockSpec((GB, D), lambda i: (i, 0)),
        ],
        out_specs=pl.BlockSpec((GB, K1), lambda i: (i, 0)),
        out_shape=jax.ShapeDtypeStruct((nb, K1), jnp.float32),
    )(weight, x)


def _tc_exp_z(raw):
    RB = 128

    def body(r_ref, e_ref, z_ref):
        i = pl.program_id(0)

        @pl.when(i == 0)
        def _():
            z_ref[...] = jnp.zeros_like(z_ref)

        e = jnp.exp(r_ref[...] * (1.0 / T))
        e_ref[...] = e
        z_ref[...] += jnp.sum(e)

    return pl.pallas_call(
        body,
        grid=(B // RB,),
        in_specs=[pl.BlockSpec((RB, K1), lambda i: (i, 0))],
        out_specs=[
            pl.BlockSpec((RB, K1), lambda i: (i, 0)),
            pl.BlockSpec((8, 128), lambda i: (0, 0)),
        ],
        out_shape=[
            jax.ShapeDtypeStruct((B, K1), jnp.float32),
            jax.ShapeDtypeStruct((8, 128), jnp.float32),
        ],
    )(raw)


def _tc_normalize(expout, zsplat):
    def body(e_ref, z_ref, o_ref):
        total = z_ref[0, 0]
        mean = total * (1.0 / (B * K1))
        zv = mean * V
        o_ref[...] = e_ref[...] / zv

    return pl.pallas_call(
        body,
        grid=(8,),
        in_specs=[
            pl.BlockSpec((B // 8, K1), lambda i: (i, 0)),
            pl.BlockSpec((8, 128), lambda i: (0, 0)),
        ],
        out_specs=pl.BlockSpec((B // 8, K1), lambda i: (i, 0)),
        out_shape=jax.ShapeDtypeStruct((B, K1), jnp.float32),
    )(expout, zsplat)


def kernel(x, memory, y, idx):
    idx = idx.at[:, 0].set(y)
    idx_flat = idx.reshape(-1)
    raws = []
    for g in range(G):
        w_g = _sc_gather(memory, idx_flat[g * BG * K1:(g + 1) * BG * K1])
        raws.append(_tc_matvec(w_g, x[g * BG:(g + 1) * BG]))
    raw = jnp.concatenate(raws, axis=0)
    expout, zsplat = _tc_exp_z(raw)
    return _tc_normalize(expout, zsplat)
